# Initial kernel scaffold; baseline (speedup 1.0000x reference)
import jax
import jax.numpy as jnp
from jax.experimental import pallas as pl


def kernel(features, classes):
    raise NotImplementedError("write your pallas kernel here")


# speedup vs baseline: 1.4537x; 1.4537x over previous
import jax
import jax.numpy as jnp
import numpy as np
from jax.experimental import pallas as pl
from jax.experimental.pallas import tpu as pltpu

_NUM_STEPS = 150
_B = 512
_STEP = np.float32(2.0 / (_NUM_STEPS - 1))
_INV_STEP = np.float32(1.0) / _STEP
_BINS_PAD = 152
_TRI_SIZE = np.float32(_B * (_B - 1) // 2)


def _cj_constants():
    t = (np.arange(_NUM_STEPS, dtype=np.float32) * _STEP
         - np.float32(1.0)).astype(np.float32)
    cj = np.zeros((_BINS_PAD, 1), np.float32)
    for j in range(_NUM_STEPS):
        cj[j, 0] = np.float32((t[j] + np.float32(1.0) - _STEP) * _INV_STEP)
    return cj


_CJ_NP = _cj_constants()


def _upper_mask():
    t = (jnp.arange(_NUM_STEPS, dtype=jnp.float32) * float(_STEP) - 1.0)[:, None]
    svals = (t + float(_STEP) / 2.0).reshape(1, _NUM_STEPS)
    s_repeat = jnp.tile(svals, (_NUM_STEPS, 1))
    delta_repeat = (jnp.floor((s_repeat + 1.0) / float(_STEP)) * float(_STEP)
                    - 1.0).astype(jnp.float32)
    indsa = delta_repeat == (t - float(_STEP))
    sub = jnp.diagonal(indsa, offset=-1)
    amask = jnp.concatenate([jnp.ones((1,), jnp.bool_), sub])
    amask = amask.astype(jnp.float32).reshape(_NUM_STEPS, 1)
    pad = jnp.zeros((_BINS_PAD - _NUM_STEPS, 1), jnp.float32)
    return jnp.concatenate([amask, pad], axis=0)


def _hist_loss_kernel(f_ref, cls_row_ref, cls_col_ref, amask_ref, cj_ref,
                      out_ref, u_s, kf_s, pm_s):
    feats = f_ref[...]
    dists = jax.lax.dot_general(
        feats, feats,
        dimension_numbers=(((1,), (1,)), ((), ())),
        preferred_element_type=jnp.float32,
    )
    u = (dists + 1.0) / _STEP
    kf = jnp.floor(u)

    row_i = jax.lax.broadcasted_iota(jnp.int32, (_B, _B), 0)
    col_i = jax.lax.broadcasted_iota(jnp.int32, (_B, _B), 1)
    tri = col_i > row_i
    eq = cls_row_ref[...] == cls_col_ref[...]

    kf_m = jnp.where(tri, kf, -1000.0)
    posm = jnp.where(tri & eq, 1.0, 0.0).astype(jnp.float32)
    pos_size = jnp.sum(posm)

    u_s[...] = u
    kf_s[...] = kf_m
    pm_s[...] = posm

    jbins = jax.lax.broadcasted_iota(
        jnp.int32, (_BINS_PAD, _B), 0).astype(jnp.float32)
    amask = amask_ref[...]
    cj = cj_ref[...]

    def body(r, carry):
        acc_tri, acc_pos = carry
        u_row = u_s[pl.ds(r, 1), :]
        kf_row = kf_s[pl.ds(r, 1), :]
        pm_row = pm_s[pl.ds(r, 1), :]
        av = u_row - cj
        bv = 2.0 - av
        eq0 = kf_row == jbins
        eq1 = kf_row == jbins - 1.0
        c = (jnp.where(eq0, bv, 0.0)
             + jnp.where(eq1, av * amask, 0.0))
        return acc_tri + c, acc_pos + c * pm_row

    acc0 = jnp.zeros((_BINS_PAD, _B), jnp.float32)
    acc_tri, acc_pos = jax.lax.fori_loop(0, _B, body, (acc0, acc0))

    pos_h = jnp.sum(acc_pos, axis=1, keepdims=True)
    neg_h = jnp.sum(acc_tri, axis=1, keepdims=True) - pos_h

    li = jax.lax.broadcasted_iota(jnp.int32, (_BINS_PAD, _BINS_PAD), 0)
    lj = jax.lax.broadcasted_iota(jnp.int32, (_BINS_PAD, _BINS_PAD), 1)
    ltri = (lj <= li).astype(jnp.float32)
    pos_cdf = jax.lax.dot_general(
        ltri, pos_h,
        dimension_numbers=(((1,), (0,)), ((), ())),
        preferred_element_type=jnp.float32,
        precision=jax.lax.Precision.HIGHEST,
    )

    neg_size = _TRI_SIZE - pos_size
    num = jnp.sum(neg_h * pos_cdf, axis=0, keepdims=True)
    out_ref[...] = num / (pos_size * neg_size)


@jax.jit
def kernel(features, classes):
    cls_row = classes.reshape(_B, 1)
    cls_col = classes.reshape(1, _B)
    amask = _upper_mask()
    cj = jnp.asarray(_CJ_NP)
    out = pl.pallas_call(
        _hist_loss_kernel,
        out_shape=jax.ShapeDtypeStruct((1, 1), jnp.float32),
        scratch_shapes=[pltpu.VMEM((_B, _B), jnp.float32)] * 3,
    )(features, cls_row, cls_col, amask, cj)
    return out[0, 0]


# speedup vs baseline: 3.8801x; 2.6692x over previous
import functools

import jax
import jax.numpy as jnp
import numpy as np
from jax import lax
from jax.experimental import pallas as pl
from jax.experimental.pallas import tpu as pltpu
from jax.experimental.pallas import tpu_sc as plsc

_NUM_STEPS = 150
_B = 512
_STEP = np.float32(2.0 / (_NUM_STEPS - 1))
_TRI_SIZE = np.float32(_B * (_B - 1) // 2)

_NW = 32
_E = _B * _B
_EPW = _E // _NW
_HIST = 1024
_NEG_BASE = 8
_POS_BASE = 520
_DUMP = 208
_UP_OFF = 257


def _upper_mask():
    t = (jnp.arange(_NUM_STEPS, dtype=jnp.float32) * float(_STEP) - 1.0)[:, None]
    svals = (t + float(_STEP) / 2.0).reshape(1, _NUM_STEPS)
    s_repeat = jnp.tile(svals, (_NUM_STEPS, 1))
    delta_repeat = (jnp.floor((s_repeat + 1.0) / float(_STEP)) * float(_STEP)
                    - 1.0).astype(jnp.float32)
    indsa = delta_repeat == (t - float(_STEP))
    sub = jnp.diagonal(indsa, offset=-1)
    amask = jnp.concatenate([jnp.ones((1,), jnp.bool_), sub])
    amask = amask.astype(jnp.float32).reshape(1, _NUM_STEPS)
    pad = jnp.zeros((1, 160 - _NUM_STEPS), jnp.float32)
    return jnp.concatenate([amask, pad], axis=1)


def _prep_kernel(f_ref, cls_row_ref, cls_col_ref, kaddr_ref, av_ref, ps_ref):
    feats = f_ref[...]
    dists = lax.dot_general(
        feats, feats,
        dimension_numbers=(((1,), (1,)), ((), ())),
        preferred_element_type=jnp.float32,
    )
    u = (dists + 1.0) / _STEP
    kf = jnp.floor(u)
    av_ref[...] = u - kf
    k_i = kf.astype(jnp.int32)

    row_i = lax.broadcasted_iota(jnp.int32, (_B, _B), 0)
    col_i = lax.broadcasted_iota(jnp.int32, (_B, _B), 1)
    tri = col_i > row_i
    eq = cls_row_ref[...] == cls_col_ref[...]
    base = jnp.where(tri,
                     jnp.where(eq, _POS_BASE, _NEG_BASE) + k_i,
                     _DUMP)
    kaddr_ref[...] = (col_i & 15) * _HIST + base
    posm = jnp.where(tri & eq, 1.0, 0.0).astype(jnp.float32)
    ps_ref[...] = jnp.sum(posm, keepdims=True)


def _sc_hist_kernel(kaddr_hbm, av_hbm, zeros_hbm, out_hbm, kv, avv, hist):
    wid = lax.axis_index("s") * 2 + lax.axis_index("c")
    base = wid * _EPW
    pltpu.sync_copy(kaddr_hbm.at[pl.ds(base, _EPW)], kv)
    pltpu.sync_copy(av_hbm.at[pl.ds(base, _EPW)], avv)
    pltpu.sync_copy(zeros_hbm, hist)

    def body(i, carry):
        k16 = kv[pl.ds(i * 16, 16)]
        a16 = avv[pl.ds(i * 16, 16)]
        plsc.addupdate_scatter(hist, [k16], 1.0 - a16)
        plsc.addupdate_scatter(hist, [k16 + _UP_OFF], a16)
        return carry

    lax.fori_loop(0, _EPW // 16, body, 0)
    pltpu.sync_copy(hist, out_hbm.at[wid])


def _finish_kernel(h_ref, amask_ref, ps_ref, out_ref):
    h = jnp.sum(h_ref[...], axis=0, keepdims=True)
    amask = amask_ref[...][:, :152]
    neg_lo = h[:, _NEG_BASE:_NEG_BASE + 152]
    neg_up = h[:, _NEG_BASE + 256:_NEG_BASE + 256 + 152]
    pos_lo = h[:, _POS_BASE:_POS_BASE + 152]
    pos_up = h[:, _POS_BASE + 256:_POS_BASE + 256 + 152]
    neg = neg_lo + neg_up * amask
    pos = pos_lo + pos_up * amask

    li = lax.broadcasted_iota(jnp.int32, (152, 152), 0)
    lj = lax.broadcasted_iota(jnp.int32, (152, 152), 1)
    m = (li <= lj).astype(jnp.float32)
    tmp = lax.dot_general(
        pos, m, dimension_numbers=(((1,), (0,)), ((), ())),
        preferred_element_type=jnp.float32,
        precision=lax.Precision.HIGHEST,
    )
    ps = ps_ref[0, 0]
    ns = _TRI_SIZE - ps
    out_ref[...] = (jnp.sum(tmp * neg, axis=1, keepdims=True)
                    / (ps * ns))


_sc_hist = functools.partial(
    pl.kernel,
    out_type=jax.ShapeDtypeStruct((_NW, 16 * _HIST), jnp.float32),
    mesh=plsc.VectorSubcoreMesh(core_axis_name="c", subcore_axis_name="s",
                                num_cores=2, num_subcores=16),
    scratch_types=[
        pltpu.VMEM((_EPW,), jnp.int32),
        pltpu.VMEM((_EPW,), jnp.float32),
        pltpu.VMEM((16 * _HIST,), jnp.float32),
    ],
    compiler_params=pltpu.CompilerParams(needs_layout_passes=False),
)(_sc_hist_kernel)


@jax.jit
def kernel(features, classes):
    cls_row = classes.reshape(_B, 1)
    cls_col = classes.reshape(1, _B)
    kaddr, av, ps = pl.pallas_call(
        _prep_kernel,
        out_shape=(
            jax.ShapeDtypeStruct((_B, _B), jnp.int32),
            jax.ShapeDtypeStruct((_B, _B), jnp.float32),
            jax.ShapeDtypeStruct((1, 1), jnp.float32),
        ),
    )(features, cls_row, cls_col)

    zeros = jnp.zeros((16 * _HIST,), jnp.float32)
    hists = _sc_hist(kaddr.reshape(_E), av.reshape(_E), zeros)

    out = pl.pallas_call(
        _finish_kernel,
        out_shape=jax.ShapeDtypeStruct((1, 1), jnp.float32),
    )(hists.reshape(_NW * 16, _HIST), _upper_mask(), ps)
    return out[0, 0]
